# baseline (speedup 1.0000x reference)
import functools

import jax
import jax.numpy as jnp
from jax import lax
from jax.experimental import pallas as pl
from jax.experimental.pallas import tpu as pltpu
from jax.experimental.pallas import tpu_sc as plsc

_B = 16384
_F = 26
_D = 16
_CB = 128


def _make_kernel():
    info = plsc.get_sparse_core_info()
    nc, ns = info.num_cores, info.num_subcores
    nw = nc * ns
    b_per_w = _B // nw
    n_chunks = b_per_w // _CB

    mesh = plsc.VectorSubcoreMesh(core_axis_name="c", subcore_axis_name="s")

    @functools.partial(
        pl.kernel,
        mesh=mesh,
        out_type=jax.ShapeDtypeStruct((_B, _D), jnp.float32),
        compiler_params=pltpu.CompilerParams(use_tc_tiling_on_sc=False),
        scratch_types=[
            pltpu.VMEM((_CB, 32), jnp.int32),
            pltpu.VMEM((_CB * 32, _D), jnp.float32),
            pltpu.VMEM((_CB, _D), jnp.float32),
            pltpu.SemaphoreType.DMA,
        ],
    )
    def emb_sum(idx_hbm, table_hbm, out_hbm, idx_v, rows_v, out_v, sem):
        wid = lax.axis_index("s") * nc + lax.axis_index("c")

        def chunk_body(c, carry):
            b0 = wid * b_per_w + c * _CB
            pltpu.sync_copy(idx_hbm.at[pl.ds(b0, _CB), pl.ds(0, 32)], idx_v)

            def fire_body(i, inner):
                pltpu.async_copy(
                    table_hbm.at[idx_v.at[i]],
                    rows_v.at[pl.ds(i * 32, 32)],
                    sem,
                )
                return inner

            lax.fori_loop(0, _CB, fire_body, 0)
            pltpu.make_async_copy(
                table_hbm.at[pl.ds(0, _CB * 32)], rows_v, sem
            ).wait()

            def reduce_body(i, inner):
                base = i * 32
                acc = rows_v[base]
                for f in range(1, _F):
                    acc = acc + rows_v[base + f]
                out_v[i] = acc
                return inner

            lax.fori_loop(0, _CB, reduce_body, 0)
            pltpu.sync_copy(out_v, out_hbm.at[pl.ds(b0, _CB)])
            return carry

        lax.fori_loop(0, n_chunks, chunk_body, 0)

    return emb_sum


def kernel(inputs, table):
    idx = jnp.pad(inputs.astype(jnp.int32), ((0, 0), (0, 128 - _F)))
    return _make_kernel()(idx, table)


# speedup vs baseline: 1.9612x; 1.9612x over previous
import functools

import jax
import jax.numpy as jnp
from jax import lax
from jax.experimental import pallas as pl
from jax.experimental.pallas import tpu as pltpu
from jax.experimental.pallas import tpu_sc as plsc

_B = 16384
_F = 26
_D = 16
_CB = 128


def _make_kernel():
    info = plsc.get_sparse_core_info()
    nc, ns = info.num_cores, info.num_subcores
    nw = nc * ns
    b_per_w = _B // nw
    n_chunks = b_per_w // _CB

    mesh = plsc.VectorSubcoreMesh(core_axis_name="c", subcore_axis_name="s")

    @functools.partial(
        pl.kernel,
        mesh=mesh,
        out_type=jax.ShapeDtypeStruct((_B, _D), jnp.float32),
        compiler_params=pltpu.CompilerParams(use_tc_tiling_on_sc=False),
        scratch_types=[
            pltpu.VMEM((32, b_per_w), jnp.int32),
            pltpu.VMEM((_F * _CB, _D), jnp.float32),
            pltpu.VMEM((_CB, _D), jnp.float32),
            pltpu.SemaphoreType.DMA,
        ],
    )
    def emb_sum(idx_hbm, table_hbm, out_hbm, idx_v, rows_v, out_v, sem):
        wid = lax.axis_index("s") * nc + lax.axis_index("c")
        pltpu.sync_copy(idx_hbm.at[pl.ds(0, 32), pl.ds(wid * b_per_w, b_per_w)], idx_v)

        def chunk_body(c, carry):
            for f in range(_F):
                pltpu.async_copy(
                    table_hbm.at[idx_v.at[f, pl.ds(c * _CB, _CB)]],
                    rows_v.at[pl.ds(f * _CB, _CB)],
                    sem,
                )
            pltpu.make_async_copy(
                table_hbm.at[pl.ds(0, _F * _CB)], rows_v, sem
            ).wait()

            def reduce_body(i, inner):
                acc = rows_v[i]
                for f in range(1, _F):
                    acc = acc + rows_v[f * _CB + i]
                out_v[i] = acc
                return inner

            lax.fori_loop(0, _CB, reduce_body, 0)
            pltpu.sync_copy(out_v, out_hbm.at[pl.ds(wid * b_per_w + c * _CB, _CB)])
            return carry

        lax.fori_loop(0, n_chunks, chunk_body, 0)

    return emb_sum


def kernel(inputs, table):
    idx_t = jnp.pad(inputs.astype(jnp.int32).T, ((0, 32 - _F), (0, 0)), mode="edge")
    return _make_kernel()(idx_t, table)
